# Initial kernel scaffold; baseline (speedup 1.0000x reference)
import jax
import jax.numpy as jnp
from jax.experimental import pallas as pl


def kernel(x, edge_index, adj_vals, W1, a1, W2, ln_w, ln_b):
    raise NotImplementedError("write your pallas kernel here")


# speedup vs baseline: 4.9869x; 4.9869x over previous
import functools

import jax
import jax.numpy as jnp
from jax import lax
from jax.experimental import pallas as pl
from jax.experimental.pallas import tpu as pltpu
from jax.experimental.pallas import tpu_sc as plsc

N = 10000
E = 320000
F = 128

NC = 2
NS = 16
NW = NC * NS
EPT = E // NW
CH = 80
NCHUNK = EPT // CH
RPT = 624
TAIL = N - NS * RPT

ROW_BLK = 1000


def _tc1_body(x_ref, w1_ref, apad_ref, h1_ref, s2_ref):
    h1 = jnp.dot(x_ref[...], w1_ref[...], preferred_element_type=jnp.float32)
    h1_ref[...] = h1
    s2_ref[...] = jnp.dot(h1, apad_ref[...], preferred_element_type=jnp.float32)


def _tc2_body(a0_ref, a1_ref, w2_ref, h2_ref):
    h = jnp.maximum(a0_ref[...] + a1_ref[...], 0.0)
    h2_ref[...] = jnp.dot(h, w2_ref[...], preferred_element_type=jnp.float32)


def _tc3_body(a0_ref, a1_ref, x_ref, lnw_ref, lnb_ref, o_ref):
    h = jnp.maximum(a0_ref[...] + a1_ref[...], 0.0) + x_ref[...]
    m = jnp.mean(h, axis=-1, keepdims=True)
    cen = h - m
    var = jnp.mean(cen * cen, axis=-1, keepdims=True)
    o_ref[...] = cen * lax.rsqrt(var + 1e-5) * lnw_ref[...] + lnb_ref[...]


def _row_spec():
    return pl.BlockSpec((ROW_BLK, F), lambda i: (i, 0))


def _full_spec():
    return pl.BlockSpec((F, F), lambda i: (0, 0))


def _vec_spec():
    return pl.BlockSpec((1, F), lambda i: (0, 0))


def _scale_and_scatter(h_hbm, acc_sh, row_v, col_v, w_v, rows_v):
    pltpu.sync_copy(h_hbm.at[col_v], rows_v)

    def scale_grp(i, _):
        wv = w_v[pl.ds(i * 16, 16)]
        for l in range(16):
            wb = jnp.full((16,), wv[l], dtype=jnp.float32)
            e = i * 16 + l
            for j in range(F // 16):
                sl = pl.ds(j * 16, 16)
                rows_v[e, sl] = rows_v[e, sl] * wb
        return 0

    lax.fori_loop(0, CH // 16, scale_grp, 0)
    pltpu.sync_copy(rows_v, acc_sh.at[row_v], add=True)


def _zero_acc(zeros, acc_sh, s):
    pltpu.sync_copy(zeros.at[pl.ds(s * RPT, RPT)], acc_sh.at[pl.ds(s * RPT, RPT)])

    @pl.when(s == NS - 1)
    def _():
        pltpu.sync_copy(zeros.at[pl.ds(NS * RPT, TAIL)],
                        acc_sh.at[pl.ds(NS * RPT, TAIL)])


def _drain_acc(acc_sh, acc_out, c, s):
    pltpu.sync_copy(acc_sh.at[pl.ds(s * RPT, RPT)],
                    acc_out.at[c, pl.ds(s * RPT, RPT)])

    @pl.when(s == NS - 1)
    def _():
        pltpu.sync_copy(acc_sh.at[pl.ds(NS * RPT, TAIL)],
                        acc_out.at[c, pl.ds(NS * RPT, TAIL)])


def _sc1_body(h1, row, col, adj, st, sb, zeros, acc_out, w_out,
              st_v, sb_v, row_v, col_v, adj_v, w_v, rows_v, acc_sh):
    c = lax.axis_index("c")
    s = lax.axis_index("s")
    wid = s * NC + c
    pltpu.sync_copy(st, st_v)
    pltpu.sync_copy(sb, sb_v)
    _zero_acc(zeros, acc_sh, s)
    plsc.subcore_barrier()

    def chunk(k, _):
        base = wid * EPT + k * CH
        pltpu.sync_copy(row.at[pl.ds(base, CH)], row_v)
        pltpu.sync_copy(col.at[pl.ds(base, CH)], col_v)
        pltpu.sync_copy(adj.at[pl.ds(base, CH)], adj_v)
        for i in range(CH // 16):
            sl = pl.ds(i * 16, 16)
            sT = plsc.load_gather(st_v, [row_v[sl]])
            sB = plsc.load_gather(sb_v, [col_v[sl]])
            sc = sT + sB
            sc = jnp.maximum(sc, 0.2 * sc)
            w_v[sl] = adj_v[sl] / (1.0 + jnp.exp(-sc))
        pltpu.sync_copy(w_v, w_out.at[pl.ds(base, CH)])
        _scale_and_scatter(h1, acc_sh, row_v, col_v, w_v, rows_v)
        return 0

    lax.fori_loop(0, NCHUNK, chunk, 0)
    plsc.subcore_barrier()
    _drain_acc(acc_sh, acc_out, c, s)


def _sc2_body(h2, row, col, w_in, zeros, acc_out,
              row_v, col_v, w_v, rows_v, acc_sh):
    c = lax.axis_index("c")
    s = lax.axis_index("s")
    wid = s * NC + c
    _zero_acc(zeros, acc_sh, s)
    plsc.subcore_barrier()

    def chunk(k, _):
        base = wid * EPT + k * CH
        pltpu.sync_copy(row.at[pl.ds(base, CH)], row_v)
        pltpu.sync_copy(col.at[pl.ds(base, CH)], col_v)
        pltpu.sync_copy(w_in.at[pl.ds(base, CH)], w_v)
        _scale_and_scatter(h2, acc_sh, row_v, col_v, w_v, rows_v)
        return 0

    lax.fori_loop(0, NCHUNK, chunk, 0)
    plsc.subcore_barrier()
    _drain_acc(acc_sh, acc_out, c, s)


@functools.cache
def _build():
    f32 = jnp.float32
    mesh = plsc.VectorSubcoreMesh(core_axis_name="c", subcore_axis_name="s",
                                  num_cores=NC, num_subcores=NS)

    tc1 = pl.pallas_call(
        _tc1_body,
        grid=(N // ROW_BLK,),
        in_specs=[_row_spec(), _full_spec(), _full_spec()],
        out_specs=[_row_spec(), _row_spec()],
        out_shape=[jax.ShapeDtypeStruct((N, F), f32),
                   jax.ShapeDtypeStruct((N, F), f32)],
    )

    sc_params = pltpu.CompilerParams(needs_layout_passes=False)

    sc1 = pl.kernel(
        _sc1_body,
        out_type=[jax.ShapeDtypeStruct((NC, N, F), f32),
                  jax.ShapeDtypeStruct((E,), f32)],
        mesh=mesh,
        compiler_params=sc_params,
        scratch_types=[
            pltpu.VMEM((N,), f32),
            pltpu.VMEM((N,), f32),
            pltpu.VMEM((CH,), jnp.int32),
            pltpu.VMEM((CH,), jnp.int32),
            pltpu.VMEM((CH,), f32),
            pltpu.VMEM((CH,), f32),
            pltpu.VMEM((CH, F), f32),
            pltpu.VMEM_SHARED((N, F), f32),
        ],
    )

    tc2 = pl.pallas_call(
        _tc2_body,
        grid=(N // ROW_BLK,),
        in_specs=[_row_spec(), _row_spec(), _full_spec()],
        out_specs=_row_spec(),
        out_shape=jax.ShapeDtypeStruct((N, F), f32),
    )

    sc2 = pl.kernel(
        _sc2_body,
        out_type=jax.ShapeDtypeStruct((NC, N, F), f32),
        mesh=mesh,
        compiler_params=sc_params,
        scratch_types=[
            pltpu.VMEM((CH,), jnp.int32),
            pltpu.VMEM((CH,), jnp.int32),
            pltpu.VMEM((CH,), f32),
            pltpu.VMEM((CH, F), f32),
            pltpu.VMEM_SHARED((N, F), f32),
        ],
    )

    tc3 = pl.pallas_call(
        _tc3_body,
        grid=(N // ROW_BLK,),
        in_specs=[_row_spec(), _row_spec(), _row_spec(), _vec_spec(), _vec_spec()],
        out_specs=_row_spec(),
        out_shape=jax.ShapeDtypeStruct((N, F), f32),
    )

    return tc1, sc1, tc2, sc2, tc3


def kernel(x, edge_index, adj_vals, W1, a1, W2, ln_w, ln_b):
    tc1, sc1, tc2, sc2, tc3 = _build()
    f32 = jnp.float32
    row = edge_index[0]
    col = edge_index[1]
    a_flat = a1[:, 0]
    a_pad = jnp.zeros((F, F), f32).at[:, 0].set(a_flat[:F]).at[:, 1].set(a_flat[F:])
    zeros = jnp.zeros((N, F), f32)

    h1, s2 = tc1(x, W1, a_pad)
    st = s2[:, 0]
    sb = s2[:, 1]
    acc, w = sc1(h1, row, col, adj_vals, st, sb, zeros)
    h2 = tc2(acc[0], acc[1], W2)
    acc2 = sc2(h2, row, col, w, zeros)
    out = tc3(acc2[0], acc2[1], x, ln_w.reshape(1, F), ln_b.reshape(1, F))
    return out


# speedup vs baseline: 11.4966x; 2.3054x over previous
import functools

import jax
import jax.numpy as jnp
from jax import lax
from jax.experimental import pallas as pl
from jax.experimental.pallas import tpu as pltpu
from jax.experimental.pallas import tpu_sc as plsc

N = 10000
E = 320000
F = 128

NC = 2
NS = 16
NW = NC * NS
EPT = E // NW
CH = 80
NCHUNK = EPT // CH
RPT = 624
TAIL = N - NS * RPT

ROW_BLK = 1000


def _tc1_body(x_ref, w1_ref, apad_ref, h1_ref, s2_ref):
    h1 = jnp.dot(x_ref[...], w1_ref[...], preferred_element_type=jnp.float32)
    h1_ref[...] = h1
    s2_ref[...] = jnp.dot(h1, apad_ref[...], preferred_element_type=jnp.float32)


def _tc2_body(a0_ref, a1_ref, w2_ref, h2_ref):
    h = jnp.maximum(a0_ref[...] + a1_ref[...], 0.0)
    h2_ref[...] = jnp.dot(h, w2_ref[...], preferred_element_type=jnp.float32)


def _tc3_body(a0_ref, a1_ref, x_ref, lnw_ref, lnb_ref, o_ref):
    h = jnp.maximum(a0_ref[...] + a1_ref[...], 0.0) + x_ref[...]
    m = jnp.mean(h, axis=-1, keepdims=True)
    cen = h - m
    var = jnp.mean(cen * cen, axis=-1, keepdims=True)
    o_ref[...] = cen * lax.rsqrt(var + 1e-5) * lnw_ref[...] + lnb_ref[...]


def _row_spec():
    return pl.BlockSpec((ROW_BLK, F), lambda i: (i, 0))


def _full_spec():
    return pl.BlockSpec((F, F), lambda i: (0, 0))


def _vec_spec():
    return pl.BlockSpec((1, F), lambda i: (0, 0))


def _scatter_pipeline(h_hbm, acc_sh, packed_v, w_v, buf_a, buf_b):
    def copy_idx(kk, ridx, cidx):
        for i in range(CH // 16):
            sl = pl.ds(i * 16, 16)
            p = packed_v[pl.ds(kk * CH + i * 16, 16)]
            ridx[sl] = jnp.bitwise_and(p, 0xFFFF)
            cidx[sl] = lax.shift_right_logical(p, 16)

    def scale(kk, rows):
        def grp(i, _):
            wv = w_v[pl.ds(kk * CH + i * 16, 16)]
            for l in range(16):
                wb = jnp.full((16,), wv[l], dtype=jnp.float32)
                e = i * 16 + l
                for j in range(F // 16):
                    sl = pl.ds(j * 16, 16)
                    rows[e, sl] = rows[e, sl] * wb
            return 0

        lax.fori_loop(0, CH // 16, grp, 0)

    def step(k, cur, nxt, first, guard):
        rows_c, ridx_c, cidx_c, semg_c, sems_c = cur
        rows_n, ridx_n, cidx_n, semg_n, sems_n = nxt

        def prefetch():
            if not first:
                pltpu.make_async_copy(rows_n, acc_sh.at[ridx_n], sems_n).wait()
            copy_idx(k + 1, ridx_n, cidx_n)
            pltpu.async_copy(h_hbm.at[cidx_n], rows_n, semg_n)

        if guard is None:
            prefetch()
        else:
            pl.when(guard)(prefetch)
        pltpu.make_async_copy(h_hbm.at[cidx_c], rows_c, semg_c).wait()
        scale(k, rows_c)
        pltpu.async_copy(rows_c, acc_sh.at[ridx_c], sems_c, add=True)

    copy_idx(0, buf_a[1], buf_a[2])
    pltpu.async_copy(h_hbm.at[buf_a[2]], buf_a[0], buf_a[3])
    step(0, buf_a, buf_b, True, None)

    def pair(t, _):
        k0 = 1 + 2 * t
        step(k0, buf_b, buf_a, False, None)
        step(k0 + 1, buf_a, buf_b, False, k0 + 2 < NCHUNK)
        return 0

    lax.fori_loop(0, (NCHUNK - 1) // 2, pair, 0)
    pltpu.make_async_copy(buf_b[0], acc_sh.at[buf_b[1]], buf_b[4]).wait()
    pltpu.make_async_copy(buf_a[0], acc_sh.at[buf_a[1]], buf_a[4]).wait()


def _zero_acc(zeros, acc_sh, s):
    pltpu.sync_copy(zeros.at[pl.ds(s * RPT, RPT)], acc_sh.at[pl.ds(s * RPT, RPT)])

    @pl.when(s == NS - 1)
    def _():
        pltpu.sync_copy(zeros.at[pl.ds(NS * RPT, TAIL)],
                        acc_sh.at[pl.ds(NS * RPT, TAIL)])


def _drain_acc(acc_sh, acc_out, c, s):
    pltpu.sync_copy(acc_sh.at[pl.ds(s * RPT, RPT)],
                    acc_out.at[c, pl.ds(s * RPT, RPT)])

    @pl.when(s == NS - 1)
    def _():
        pltpu.sync_copy(acc_sh.at[pl.ds(NS * RPT, TAIL)],
                        acc_out.at[c, pl.ds(NS * RPT, TAIL)])


def _scatter_sync(h_hbm, acc_sh, packed_v, w_v, buf_a, buf_b):
    rows, ridx, cidx, _, _ = buf_a

    def chunk(kk, _):
        for i in range(CH // 16):
            sl = pl.ds(i * 16, 16)
            p = packed_v[pl.ds(kk * CH + i * 16, 16)]
            ridx[sl] = jnp.bitwise_and(p, 0xFFFF)
            cidx[sl] = lax.shift_right_logical(p, 16)
        pltpu.sync_copy(h_hbm.at[cidx], rows)

        def grp(i, _):
            wv = w_v[pl.ds(kk * CH + i * 16, 16)]
            for l in range(16):
                wb = jnp.full((16,), wv[l], dtype=jnp.float32)
                e = i * 16 + l
                for j in range(F // 16):
                    sl = pl.ds(j * 16, 16)
                    rows[e, sl] = rows[e, sl] * wb
            return 0

        lax.fori_loop(0, CH // 16, grp, 0)
        pltpu.sync_copy(rows, acc_sh.at[ridx], add=True)
        return 0

    lax.fori_loop(0, NCHUNK, chunk, 0)


def _scw_body(packed, adj, st, sb, w_out,
              packed_v, adj_v, st_v, sb_v, w_v):
    c = lax.axis_index("c")
    s = lax.axis_index("s")
    wid = s * NC + c
    ebase = wid * EPT
    pltpu.sync_copy(st, st_v)
    pltpu.sync_copy(sb, sb_v)
    pltpu.sync_copy(packed.at[pl.ds(ebase, EPT)], packed_v)
    pltpu.sync_copy(adj.at[pl.ds(ebase, EPT)], adj_v)

    def wbody(i, _):
        sl = pl.ds(i * 16, 16)
        p = packed_v[sl]
        sT = plsc.load_gather(st_v, [jnp.bitwise_and(p, 0xFFFF)])
        sB = plsc.load_gather(sb_v, [lax.shift_right_logical(p, 16)])
        sc = sT + sB
        sc = jnp.maximum(sc, 0.2 * sc)
        w_v[sl] = adj_v[sl] / (1.0 + jnp.exp(-sc))
        return 0

    lax.fori_loop(0, EPT // 16, wbody, 0)
    pltpu.sync_copy(w_v, w_out.at[pl.ds(ebase, EPT)])


def _scat_body(h, packed, w_in, zeros, acc_out,
               packed_v, w_v,
               rows0, rows1, ridx0, ridx1, cidx0, cidx1, acc_sh,
               semg0, semg1, sems0, sems1):
    c = lax.axis_index("c")
    s = lax.axis_index("s")
    wid = s * NC + c
    ebase = wid * EPT
    pltpu.sync_copy(packed.at[pl.ds(ebase, EPT)], packed_v)
    pltpu.sync_copy(w_in.at[pl.ds(ebase, EPT)], w_v)
    _zero_acc(zeros, acc_sh, s)
    plsc.subcore_barrier()
    _scatter_pipeline(h, acc_sh, packed_v, w_v,
                      (rows0, ridx0, cidx0, semg0, sems0),
                      (rows1, ridx1, cidx1, semg1, sems1))
    plsc.subcore_barrier()
    _drain_acc(acc_sh, acc_out, c, s)


@functools.cache
def _build():
    f32 = jnp.float32
    mesh = plsc.VectorSubcoreMesh(core_axis_name="c", subcore_axis_name="s",
                                  num_cores=NC, num_subcores=NS)

    tc1 = pl.pallas_call(
        _tc1_body,
        grid=(N // ROW_BLK,),
        in_specs=[_row_spec(), _full_spec(), _full_spec()],
        out_specs=[_row_spec(), _row_spec()],
        out_shape=[jax.ShapeDtypeStruct((N, F), f32),
                   jax.ShapeDtypeStruct((N, F), f32)],
    )

    sc_params = pltpu.CompilerParams(needs_layout_passes=False)

    scw = pl.kernel(
        _scw_body,
        out_type=jax.ShapeDtypeStruct((E,), f32),
        mesh=mesh,
        compiler_params=sc_params,
        scratch_types=[
            pltpu.VMEM((EPT,), jnp.int32),
            pltpu.VMEM((EPT,), f32),
            pltpu.VMEM((N,), f32),
            pltpu.VMEM((N,), f32),
            pltpu.VMEM((EPT,), f32),
        ],
    )

    tc2 = pl.pallas_call(
        _tc2_body,
        grid=(N // ROW_BLK,),
        in_specs=[_row_spec(), _row_spec(), _full_spec()],
        out_specs=_row_spec(),
        out_shape=jax.ShapeDtypeStruct((N, F), f32),
    )

    scat = pl.kernel(
        _scat_body,
        out_type=jax.ShapeDtypeStruct((NC, N, F), f32),
        mesh=mesh,
        compiler_params=sc_params,
        scratch_types=[
            pltpu.VMEM((EPT,), jnp.int32),
            pltpu.VMEM((EPT,), f32),
            pltpu.VMEM((CH, F), f32),
            pltpu.VMEM((CH, F), f32),
            pltpu.VMEM((CH,), jnp.int32),
            pltpu.VMEM((CH,), jnp.int32),
            pltpu.VMEM((CH,), jnp.int32),
            pltpu.VMEM((CH,), jnp.int32),
            pltpu.VMEM_SHARED((N, F), f32),
            pltpu.SemaphoreType.DMA,
            pltpu.SemaphoreType.DMA,
            pltpu.SemaphoreType.DMA,
            pltpu.SemaphoreType.DMA,
        ],
    )

    tc3 = pl.pallas_call(
        _tc3_body,
        grid=(N // ROW_BLK,),
        in_specs=[_row_spec(), _row_spec(), _row_spec(), _vec_spec(), _vec_spec()],
        out_specs=_row_spec(),
        out_shape=jax.ShapeDtypeStruct((N, F), f32),
    )

    return tc1, scw, scat, tc2, tc3


def kernel(x, edge_index, adj_vals, W1, a1, W2, ln_w, ln_b):
    tc1, scw, scat, tc2, tc3 = _build()
    f32 = jnp.float32
    row = edge_index[0]
    col = edge_index[1]
    packed = jnp.bitwise_or(row, jnp.left_shift(col, 16))
    a_flat = a1[:, 0]
    a_pad = jnp.zeros((F, F), f32).at[:, 0].set(a_flat[:F]).at[:, 1].set(a_flat[F:])
    zeros = jnp.zeros((N, F), f32)

    h1, s2 = tc1(x, W1, a_pad)
    st = s2[:, 0]
    sb = s2[:, 1]
    w = scw(packed, adj_vals, st, sb)
    acc = scat(h1, packed, w, zeros)
    h2 = tc2(acc[0], acc[1], W2)
    acc2 = scat(h2, packed, w, zeros)
    out = tc3(acc2[0], acc2[1], x, ln_w.reshape(1, F), ln_b.reshape(1, F))
    return out


# speedup vs baseline: 12.6886x; 1.1037x over previous
import functools

import jax
import jax.numpy as jnp
from jax import lax
from jax.experimental import pallas as pl
from jax.experimental.pallas import tpu as pltpu
from jax.experimental.pallas import tpu_sc as plsc

N = 10000
E = 320000
F = 128

NC = 2
NS = 16
NW = NC * NS
EPT = E // NW
CH = 80
NCHUNK = EPT // CH
RPT = 624
TAIL = N - NS * RPT

ROW_BLK = 1000


def _tc1_body(x_ref, w1_ref, apad_ref, h1_ref, s2_ref):
    h1 = jnp.dot(x_ref[...], w1_ref[...], preferred_element_type=jnp.float32)
    h1_ref[...] = h1
    s2_ref[...] = jnp.dot(h1, apad_ref[...], preferred_element_type=jnp.float32)


def _tc2_body(a0_ref, a1_ref, w2_ref, h2_ref):
    h = jnp.maximum(a0_ref[...] + a1_ref[...], 0.0)
    h2_ref[...] = jnp.dot(h, w2_ref[...], preferred_element_type=jnp.float32)


def _tc3_body(a0_ref, a1_ref, x_ref, lnw_ref, lnb_ref, o_ref):
    h = jnp.maximum(a0_ref[...] + a1_ref[...], 0.0) + x_ref[...]
    m = jnp.mean(h, axis=-1, keepdims=True)
    cen = h - m
    var = jnp.mean(cen * cen, axis=-1, keepdims=True)
    o_ref[...] = cen * lax.rsqrt(var + 1e-5) * lnw_ref[...] + lnb_ref[...]


def _row_spec():
    return pl.BlockSpec((ROW_BLK, F), lambda i: (i, 0))


def _full_spec():
    return pl.BlockSpec((F, F), lambda i: (0, 0))


def _vec_spec():
    return pl.BlockSpec((1, F), lambda i: (0, 0))


def _scatter_pipeline(h_hbm, acc_sh, packed_hbm, ebase, w_v, sets):
    def issue_p(kk, S):
        pltpu.async_copy(packed_hbm.at[pl.ds(ebase + kk * CH, CH)], S[1], S[4])

    def wait_p(S):
        pltpu.make_async_copy(packed_hbm.at[pl.ds(ebase, CH)], S[1], S[4]).wait()

    def unpack(S):
        for i in range(CH // 16):
            sl = pl.ds(i * 16, 16)
            p = S[1][sl]
            S[2][sl] = jnp.bitwise_and(p, 0xFFFF)
            S[3][sl] = lax.shift_right_logical(p, 16)

    def issue_g(S):
        pltpu.async_copy(h_hbm.at[S[3]], S[0], S[5])

    def wait_g(S):
        pltpu.make_async_copy(h_hbm.at[S[3]], S[0], S[5]).wait()

    def issue_a(S):
        pltpu.async_copy(S[0], acc_sh.at[S[2]], S[6], add=True)

    def wait_a(S):
        pltpu.make_async_copy(S[0], acc_sh.at[S[2]], S[6]).wait()

    def scale(kk, rows):
        def grp(i, _):
            wv = w_v[pl.ds(kk * CH + i * 16, 16)]
            for l in range(16):
                wb = jnp.full((16,), wv[l], dtype=jnp.float32)
                e = i * 16 + l
                for j in range(F // 16):
                    sl = pl.ds(j * 16, 16)
                    rows[e, sl] = rows[e, sl] * wb
            return 0

        lax.fori_loop(0, CH // 16, grp, 0)

    def step(k, s0, s1, s2, wait_prev_a):
        @pl.when(k + 1 < NCHUNK)
        def _():
            if wait_prev_a:
                wait_a(s1)
            wait_p(s1)
            unpack(s1)
            issue_g(s1)

        @pl.when(k + 2 < NCHUNK)
        def _():
            issue_p(k + 2, s2)

        wait_g(s0)
        scale(k, s0[0])
        issue_a(s0)

    issue_p(0, sets[0])
    issue_p(1, sets[1])
    wait_p(sets[0])
    unpack(sets[0])
    issue_g(sets[0])
    step(0, sets[0], sets[1], sets[2], False)
    step(1, sets[1], sets[2], sets[0], False)

    def triple(t, _):
        k = 3 * t + 2
        step(k, sets[2], sets[0], sets[1], True)
        step(k + 1, sets[0], sets[1], sets[2], True)
        step(k + 2, sets[1], sets[2], sets[0], True)
        return 0

    lax.fori_loop(0, (NCHUNK - 2) // 3, triple, 0)
    wait_a(sets[(NCHUNK - 3) % 3])
    wait_a(sets[(NCHUNK - 2) % 3])
    wait_a(sets[(NCHUNK - 1) % 3])


def _zero_acc(zeros, acc_sh, s):
    pltpu.sync_copy(zeros.at[pl.ds(s * RPT, RPT)], acc_sh.at[pl.ds(s * RPT, RPT)])

    @pl.when(s == NS - 1)
    def _():
        pltpu.sync_copy(zeros.at[pl.ds(NS * RPT, TAIL)],
                        acc_sh.at[pl.ds(NS * RPT, TAIL)])


def _drain_acc(acc_sh, acc_out, c, s):
    pltpu.sync_copy(acc_sh.at[pl.ds(s * RPT, RPT)],
                    acc_out.at[c, pl.ds(s * RPT, RPT)])

    @pl.when(s == NS - 1)
    def _():
        pltpu.sync_copy(acc_sh.at[pl.ds(NS * RPT, TAIL)],
                        acc_out.at[c, pl.ds(NS * RPT, TAIL)])


def _scw_body(packed, adj, st, sb, w_out,
              packed_v, adj_v, st_v, sb_v, w_v):
    c = lax.axis_index("c")
    s = lax.axis_index("s")
    wid = s * NC + c
    ebase = wid * EPT
    pltpu.sync_copy(st, st_v)
    pltpu.sync_copy(sb, sb_v)
    pltpu.sync_copy(packed.at[pl.ds(ebase, EPT)], packed_v)
    pltpu.sync_copy(adj.at[pl.ds(ebase, EPT)], adj_v)

    def wbody(i, _):
        sl = pl.ds(i * 16, 16)
        p = packed_v[sl]
        sT = plsc.load_gather(st_v, [jnp.bitwise_and(p, 0xFFFF)])
        sB = plsc.load_gather(sb_v, [lax.shift_right_logical(p, 16)])
        sc = sT + sB
        sc = jnp.maximum(sc, 0.2 * sc)
        w_v[sl] = adj_v[sl] / (1.0 + jnp.exp(-sc))
        return 0

    lax.fori_loop(0, EPT // 16, wbody, 0)
    pltpu.sync_copy(w_v, w_out.at[pl.ds(ebase, EPT)])


def _scat_body(h, packed, w_in, zeros, acc_out,
               w_v,
               rows0, rows1, rows2, pbuf0, pbuf1, pbuf2,
               ridx0, ridx1, ridx2, cidx0, cidx1, cidx2, acc_sh,
               semp0, semp1, semp2, semg0, semg1, semg2,
               sema0, sema1, sema2):
    c = lax.axis_index("c")
    s = lax.axis_index("s")
    wid = s * NC + c
    ebase = wid * EPT
    pltpu.sync_copy(w_in.at[pl.ds(ebase, EPT)], w_v)
    _zero_acc(zeros, acc_sh, s)
    plsc.subcore_barrier()
    sets = ((rows0, pbuf0, ridx0, cidx0, semp0, semg0, sema0),
            (rows1, pbuf1, ridx1, cidx1, semp1, semg1, sema1),
            (rows2, pbuf2, ridx2, cidx2, semp2, semg2, sema2))
    _scatter_pipeline(h, acc_sh, packed, ebase, w_v, sets)
    plsc.subcore_barrier()
    _drain_acc(acc_sh, acc_out, c, s)


@functools.cache
def _build():
    f32 = jnp.float32
    mesh = plsc.VectorSubcoreMesh(core_axis_name="c", subcore_axis_name="s",
                                  num_cores=NC, num_subcores=NS)

    tc1 = pl.pallas_call(
        _tc1_body,
        grid=(N // ROW_BLK,),
        in_specs=[_row_spec(), _full_spec(), _full_spec()],
        out_specs=[_row_spec(), _row_spec()],
        out_shape=[jax.ShapeDtypeStruct((N, F), f32),
                   jax.ShapeDtypeStruct((N, F), f32)],
    )

    sc_params = pltpu.CompilerParams(needs_layout_passes=False)

    scw = pl.kernel(
        _scw_body,
        out_type=jax.ShapeDtypeStruct((E,), f32),
        mesh=mesh,
        compiler_params=sc_params,
        scratch_types=[
            pltpu.VMEM((EPT,), jnp.int32),
            pltpu.VMEM((EPT,), f32),
            pltpu.VMEM((N,), f32),
            pltpu.VMEM((N,), f32),
            pltpu.VMEM((EPT,), f32),
        ],
    )

    tc2 = pl.pallas_call(
        _tc2_body,
        grid=(N // ROW_BLK,),
        in_specs=[_row_spec(), _row_spec(), _full_spec()],
        out_specs=_row_spec(),
        out_shape=jax.ShapeDtypeStruct((N, F), f32),
    )

    scat = pl.kernel(
        _scat_body,
        out_type=jax.ShapeDtypeStruct((NC, N, F), f32),
        mesh=mesh,
        compiler_params=sc_params,
        scratch_types=(
            [pltpu.VMEM((EPT,), f32)]
            + [pltpu.VMEM((CH, F), f32)] * 3
            + [pltpu.VMEM((CH,), jnp.int32)] * 3
            + [pltpu.VMEM((CH,), jnp.int32)] * 3
            + [pltpu.VMEM((CH,), jnp.int32)] * 3
            + [pltpu.VMEM_SHARED((N, F), f32)]
            + [pltpu.SemaphoreType.DMA] * 9
        ),
    )

    tc3 = pl.pallas_call(
        _tc3_body,
        grid=(N // ROW_BLK,),
        in_specs=[_row_spec(), _row_spec(), _row_spec(), _vec_spec(), _vec_spec()],
        out_specs=_row_spec(),
        out_shape=jax.ShapeDtypeStruct((N, F), f32),
    )

    return tc1, scw, scat, tc2, tc3


def kernel(x, edge_index, adj_vals, W1, a1, W2, ln_w, ln_b):
    tc1, scw, scat, tc2, tc3 = _build()
    f32 = jnp.float32
    row = edge_index[0]
    col = edge_index[1]
    packed = jnp.bitwise_or(row, jnp.left_shift(col, 16))
    a_flat = a1[:, 0]
    a_pad = jnp.zeros((F, F), f32).at[:, 0].set(a_flat[:F]).at[:, 1].set(a_flat[F:])
    zeros = jnp.zeros((N, F), f32)

    h1, s2 = tc1(x, W1, a_pad)
    st = s2[:, 0]
    sb = s2[:, 1]
    w = scw(packed, adj_vals, st, sb)
    acc = scat(h1, packed, w, zeros)
    h2 = tc2(acc[0], acc[1], W2)
    acc2 = scat(h2, packed, w, zeros)
    out = tc3(acc2[0], acc2[1], x, ln_w.reshape(1, F), ln_b.reshape(1, F))
    return out


# speedup vs baseline: 12.8663x; 1.0140x over previous
import functools

import jax
import jax.numpy as jnp
from jax import lax
from jax.experimental import pallas as pl
from jax.experimental.pallas import tpu as pltpu
from jax.experimental.pallas import tpu_sc as plsc

N = 10000
E = 320000
F = 128

NC = 2
NS = 16
NW = NC * NS
EPT = E // NW
CH = 80
NCHUNK = EPT // CH
RPT = 624
TAIL = N - NS * RPT

ROW_BLK = 1000


def _tc1a_body(x_ref, w1_ref, apad_ref, s2_ref):
    v = jnp.dot(w1_ref[...], apad_ref[...], preferred_element_type=jnp.float32)
    s2_ref[...] = jnp.dot(x_ref[...], v, preferred_element_type=jnp.float32)


def _tc1b_body(x_ref, w1_ref, h1_ref):
    h1_ref[...] = jnp.dot(x_ref[...], w1_ref[...],
                          preferred_element_type=jnp.float32)


def _tc2_body(a0_ref, a1_ref, w2_ref, h2_ref):
    h = jnp.maximum(a0_ref[...] + a1_ref[...], 0.0)
    h2_ref[...] = jnp.dot(h, w2_ref[...], preferred_element_type=jnp.float32)


def _tc3_body(a0_ref, a1_ref, x_ref, lnw_ref, lnb_ref, o_ref):
    h = jnp.maximum(a0_ref[...] + a1_ref[...], 0.0) + x_ref[...]
    m = jnp.mean(h, axis=-1, keepdims=True)
    cen = h - m
    var = jnp.mean(cen * cen, axis=-1, keepdims=True)
    o_ref[...] = cen * lax.rsqrt(var + 1e-5) * lnw_ref[...] + lnb_ref[...]


def _row_spec():
    return pl.BlockSpec((ROW_BLK, F), lambda i: (i, 0))


def _full_spec():
    return pl.BlockSpec((F, F), lambda i: (0, 0))


def _vec_spec():
    return pl.BlockSpec((1, F), lambda i: (0, 0))


def _scatter_pipeline(h_hbm, acc_sh, packed_hbm, ebase, w_v, sets):
    def issue_p(kk, S):
        pltpu.async_copy(packed_hbm.at[pl.ds(ebase + kk * CH, CH)], S[1], S[4])

    def wait_p(S):
        pltpu.make_async_copy(packed_hbm.at[pl.ds(ebase, CH)], S[1], S[4]).wait()

    def unpack(S):
        for i in range(CH // 16):
            sl = pl.ds(i * 16, 16)
            p = S[1][sl]
            S[2][sl] = jnp.bitwise_and(p, 0xFFFF)
            S[3][sl] = lax.shift_right_logical(p, 16)

    def issue_g(S):
        pltpu.async_copy(h_hbm.at[S[3]], S[0], S[5])

    def wait_g(S):
        pltpu.make_async_copy(h_hbm.at[S[3]], S[0], S[5]).wait()

    def issue_a(S):
        pltpu.async_copy(S[0], acc_sh.at[S[2]], S[6], add=True)

    def wait_a(S):
        pltpu.make_async_copy(S[0], acc_sh.at[S[2]], S[6]).wait()

    def scale(kk, rows):
        def grp(i, _):
            wv = w_v[pl.ds(kk * CH + i * 16, 16)]
            for l in range(16):
                wb = jnp.full((16,), wv[l], dtype=jnp.float32)
                e = i * 16 + l
                for j in range(F // 16):
                    sl = pl.ds(j * 16, 16)
                    rows[e, sl] = rows[e, sl] * wb
            return 0

        lax.fori_loop(0, CH // 16, grp, 0)

    def step(k, s0, s1, s2, wait_prev_a):
        @pl.when(k + 1 < NCHUNK)
        def _():
            if wait_prev_a:
                wait_a(s1)
            wait_p(s1)
            unpack(s1)
            issue_g(s1)

        @pl.when(k + 2 < NCHUNK)
        def _():
            issue_p(k + 2, s2)

        wait_g(s0)
        scale(k, s0[0])
        issue_a(s0)

    issue_p(0, sets[0])
    issue_p(1, sets[1])
    wait_p(sets[0])
    unpack(sets[0])
    issue_g(sets[0])
    step(0, sets[0], sets[1], sets[2], False)
    step(1, sets[1], sets[2], sets[0], False)

    def triple(t, _):
        k = 3 * t + 2
        step(k, sets[2], sets[0], sets[1], True)
        step(k + 1, sets[0], sets[1], sets[2], True)
        step(k + 2, sets[1], sets[2], sets[0], True)
        return 0

    lax.fori_loop(0, (NCHUNK - 2) // 3, triple, 0)
    wait_a(sets[(NCHUNK - 3) % 3])
    wait_a(sets[(NCHUNK - 2) % 3])
    wait_a(sets[(NCHUNK - 1) % 3])


def _zero_acc(zeros, acc_sh, s):
    pltpu.sync_copy(zeros.at[pl.ds(s * RPT, RPT)], acc_sh.at[pl.ds(s * RPT, RPT)])

    @pl.when(s == NS - 1)
    def _():
        pltpu.sync_copy(zeros.at[pl.ds(NS * RPT, TAIL)],
                        acc_sh.at[pl.ds(NS * RPT, TAIL)])


def _drain_acc(acc_sh, acc_out, c, s):
    pltpu.sync_copy(acc_sh.at[pl.ds(s * RPT, RPT)],
                    acc_out.at[c, pl.ds(s * RPT, RPT)])

    @pl.when(s == NS - 1)
    def _():
        pltpu.sync_copy(acc_sh.at[pl.ds(NS * RPT, TAIL)],
                        acc_out.at[c, pl.ds(NS * RPT, TAIL)])


def _scw_body(packed, adj, st, sb, w_out,
              packed_v, adj_v, st_v, sb_v, w_v,
              sem0, sem1, sem2, sem3):
    c = lax.axis_index("c")
    s = lax.axis_index("s")
    wid = s * NC + c
    ebase = wid * EPT
    d0 = pltpu.async_copy(st, st_v, sem0)
    d1 = pltpu.async_copy(sb, sb_v, sem1)
    d2 = pltpu.async_copy(packed.at[pl.ds(ebase, EPT)], packed_v, sem2)
    d3 = pltpu.async_copy(adj.at[pl.ds(ebase, EPT)], adj_v, sem3)
    d0.wait()
    d1.wait()
    d2.wait()
    d3.wait()

    def wbody(i, _):
        sl = pl.ds(i * 16, 16)
        p = packed_v[sl]
        sT = plsc.load_gather(st_v, [jnp.bitwise_and(p, 0xFFFF)])
        sB = plsc.load_gather(sb_v, [lax.shift_right_logical(p, 16)])
        sc = sT + sB
        sc = jnp.maximum(sc, 0.2 * sc)
        w_v[sl] = adj_v[sl] / (1.0 + jnp.exp(-sc))
        return 0

    lax.fori_loop(0, EPT // 16, wbody, 0)
    pltpu.sync_copy(w_v, w_out.at[pl.ds(ebase, EPT)])


def _scat_body(h, packed, w_in, zeros, acc_out,
               w_v,
               rows0, rows1, rows2, pbuf0, pbuf1, pbuf2,
               ridx0, ridx1, ridx2, cidx0, cidx1, cidx2, acc_sh,
               semp0, semp1, semp2, semg0, semg1, semg2,
               sema0, sema1, sema2):
    c = lax.axis_index("c")
    s = lax.axis_index("s")
    wid = s * NC + c
    ebase = wid * EPT
    pltpu.sync_copy(w_in.at[pl.ds(ebase, EPT)], w_v)
    _zero_acc(zeros, acc_sh, s)
    plsc.subcore_barrier()
    sets = ((rows0, pbuf0, ridx0, cidx0, semp0, semg0, sema0),
            (rows1, pbuf1, ridx1, cidx1, semp1, semg1, sema1),
            (rows2, pbuf2, ridx2, cidx2, semp2, semg2, sema2))
    _scatter_pipeline(h, acc_sh, packed, ebase, w_v, sets)
    plsc.subcore_barrier()
    _drain_acc(acc_sh, acc_out, c, s)


@functools.cache
def _build():
    f32 = jnp.float32
    mesh = plsc.VectorSubcoreMesh(core_axis_name="c", subcore_axis_name="s",
                                  num_cores=NC, num_subcores=NS)

    tc1a = pl.pallas_call(
        _tc1a_body,
        grid=(N // ROW_BLK,),
        in_specs=[_row_spec(), _full_spec(), _full_spec()],
        out_specs=_row_spec(),
        out_shape=jax.ShapeDtypeStruct((N, F), f32),
    )

    tc1b = pl.pallas_call(
        _tc1b_body,
        grid=(N // ROW_BLK,),
        in_specs=[_row_spec(), _full_spec()],
        out_specs=_row_spec(),
        out_shape=jax.ShapeDtypeStruct((N, F), f32),
    )

    sc_params = pltpu.CompilerParams(needs_layout_passes=False)

    scw = pl.kernel(
        _scw_body,
        out_type=jax.ShapeDtypeStruct((E,), f32),
        mesh=mesh,
        compiler_params=sc_params,
        scratch_types=(
            [pltpu.VMEM((EPT,), jnp.int32),
             pltpu.VMEM((EPT,), f32),
             pltpu.VMEM((N,), f32),
             pltpu.VMEM((N,), f32),
             pltpu.VMEM((EPT,), f32)]
            + [pltpu.SemaphoreType.DMA] * 4
        ),
    )

    tc2 = pl.pallas_call(
        _tc2_body,
        grid=(N // ROW_BLK,),
        in_specs=[_row_spec(), _row_spec(), _full_spec()],
        out_specs=_row_spec(),
        out_shape=jax.ShapeDtypeStruct((N, F), f32),
    )

    scat = pl.kernel(
        _scat_body,
        out_type=jax.ShapeDtypeStruct((NC, N, F), f32),
        mesh=mesh,
        compiler_params=sc_params,
        scratch_types=(
            [pltpu.VMEM((EPT,), f32)]
            + [pltpu.VMEM((CH, F), f32)] * 3
            + [pltpu.VMEM((CH,), jnp.int32)] * 3
            + [pltpu.VMEM((CH,), jnp.int32)] * 3
            + [pltpu.VMEM((CH,), jnp.int32)] * 3
            + [pltpu.VMEM_SHARED((N, F), f32)]
            + [pltpu.SemaphoreType.DMA] * 9
        ),
    )

    tc3 = pl.pallas_call(
        _tc3_body,
        grid=(N // ROW_BLK,),
        in_specs=[_row_spec(), _row_spec(), _row_spec(), _vec_spec(), _vec_spec()],
        out_specs=_row_spec(),
        out_shape=jax.ShapeDtypeStruct((N, F), f32),
    )

    return tc1a, tc1b, scw, scat, tc2, tc3


def kernel(x, edge_index, adj_vals, W1, a1, W2, ln_w, ln_b):
    tc1a, tc1b, scw, scat, tc2, tc3 = _build()
    f32 = jnp.float32
    row = edge_index[0]
    col = edge_index[1]
    packed = jnp.bitwise_or(row, jnp.left_shift(col, 16))
    a_flat = a1[:, 0]
    a_pad = jnp.zeros((F, F), f32).at[:, 0].set(a_flat[:F]).at[:, 1].set(a_flat[F:])
    zeros = jnp.zeros((N, F), f32)

    s2 = tc1a(x, W1, a_pad)
    st = s2[:, 0]
    sb = s2[:, 1]
    w = scw(packed, adj_vals, st, sb)
    h1 = tc1b(x, W1)
    acc = scat(h1, packed, w, zeros)
    h2 = tc2(acc[0], acc[1], W2)
    acc2 = scat(h2, packed, w, zeros)
    out = tc3(acc2[0], acc2[1], x, ln_w.reshape(1, F), ln_b.reshape(1, F))
    return out
